# baseline (speedup 1.0000x reference)
import jax
import jax.numpy as jnp
from jax.experimental import pallas as pl
from jax.experimental.pallas import tpu as pltpu


_LOG2E = 1.4426950408889634
_LN2 = 0.6931471805599453


def _bce_block(x, y):
    l = jnp.log2(1.0 + jnp.exp2(jnp.abs(x) * -_LOG2E))
    return jnp.maximum(x, 0.0) - x * y + _LN2 * l


def kernel(img, msg_logits, target_msg):
    del img
    B, bits = msg_logits.shape
    inv_b = 1.0 / float(B)
    t2 = target_msg.reshape(1, bits)

    per_core = B // 2
    tb = next((t for t in (8192, 4096, 2048, 1024, 512, 256, 128, 64, 32, 16, 8)
               if B % (2 * t) == 0), None)

    if tb is not None:
        nb = per_core // tb

        def body(x_ref, t_ref, o_ref):
            j = pl.program_id(1)

            @pl.when(j == 0)
            def _():
                o_ref[...] = jnp.zeros_like(o_ref)

            x = x_ref[...]
            y = t_ref[...]
            o_ref[...] += jnp.sum(_bce_block(x, y)).reshape(1, 1, 1)

            @pl.when(j == pl.num_programs(1) - 1)
            def _():
                o_ref[...] *= jnp.float32(inv_b)

        out = pl.pallas_call(
            body,
            out_shape=jax.ShapeDtypeStruct((2, 1, 1), jnp.float32),
            grid_spec=pltpu.PrefetchScalarGridSpec(
                num_scalar_prefetch=0,
                grid=(2, nb),
                in_specs=[
                    pl.BlockSpec((tb, bits), lambda c, j, _nb=nb: (c * _nb + j, 0)),
                    pl.BlockSpec((1, bits), lambda c, j: (0, 0)),
                ],
                out_specs=pl.BlockSpec((1, 1, 1), lambda c, j: (c, 0, 0)),
            ),
            compiler_params=pltpu.CompilerParams(
                dimension_semantics=("parallel", "arbitrary")),
        )(msg_logits, t2)
        return jnp.sum(out)

    tb = B if B <= 512 else 256
    nb = pl.cdiv(B, tb)
    ragged = (B % tb) != 0

    def body1(x_ref, t_ref, o_ref):
        j = pl.program_id(0)

        @pl.when(j == 0)
        def _():
            o_ref[...] = jnp.zeros_like(o_ref)

        x = x_ref[...].astype(jnp.float32)
        y = t_ref[...].astype(jnp.float32)
        per = _bce_block(x, y)
        if ragged:
            row = jax.lax.broadcasted_iota(jnp.int32, per.shape, 0) + j * tb
            per = jnp.where(row < B, per, 0.0)
        o_ref[...] += jnp.sum(per, axis=(0, 1), keepdims=True)

        @pl.when(j == pl.num_programs(0) - 1)
        def _():
            o_ref[...] *= jnp.float32(inv_b)

    out = pl.pallas_call(
        body1,
        out_shape=jax.ShapeDtypeStruct((1, 1), jnp.float32),
        grid_spec=pltpu.PrefetchScalarGridSpec(
            num_scalar_prefetch=0,
            grid=(nb,),
            in_specs=[
                pl.BlockSpec((tb, bits), lambda j: (j, 0)),
                pl.BlockSpec((1, bits), lambda j: (0, 0)),
            ],
            out_specs=pl.BlockSpec((1, 1), lambda j: (0, 0)),
        ),
        compiler_params=pltpu.CompilerParams(
            dimension_semantics=("arbitrary",)),
    )(msg_logits, t2)
    return out[0, 0]
